# baseline (speedup 1.0000x reference)
import numpy as np
import jax
import jax.numpy as jnp
from jax import lax
from jax.experimental import pallas as pl
from jax.experimental.pallas import tpu as pltpu

_H_DISC = 4
_W_DISC = 4
_DISC = _H_DISC * _W_DISC
_SEQ_BLOCK = 8
_CH_BLOCK = 32


def _interp_matrix(n_in: int, n_out: int) -> np.ndarray:
    a = np.zeros((n_out, n_in), np.float64)
    s = n_in / n_out
    for i in range(n_out):
        x = (i + 0.5) * s - 0.5
        lo = int(np.floor(x))
        t = x - lo
        for idx, w in ((lo, 1.0 - t), (lo + 1, t)):
            a[i, min(max(idx, 0), n_in - 1)] += w
    return a.astype(np.float32)


def _upsample_kernel(emb_ref, a1_ref, a2_ref, f_ref, o_ref):
    sb, ch = emb_ref.shape[0], f_ref.shape[2]
    e4 = emb_ref[...].reshape(sb, ch, _H_DISC, _W_DISC)
    t1 = lax.dot_general(e4, a1_ref[...], (((2,), (1,)), ((), ())))
    cont = lax.dot_general(t1, a2_ref[...], (((2,), (1,)), ((), ())))
    o_ref[...] = f_ref[...] + cont[None]


def kernel(f, emb_w):
    batch, seq, ch, fh, fw = f.shape

    a1 = jnp.asarray(_interp_matrix(_H_DISC, fh))
    a2 = jnp.asarray(_interp_matrix(_W_DISC, fw))

    sb = _SEQ_BLOCK
    cb = min(_CH_BLOCK, ch)
    grid = (seq // sb, ch // cb)

    out = pl.pallas_call(
        _upsample_kernel,
        grid=grid,
        in_specs=[
            pl.BlockSpec((sb, cb * _DISC), lambda i, j: (i, j)),
            pl.BlockSpec((fh, _H_DISC), lambda i, j: (0, 0)),
            pl.BlockSpec((fw, _W_DISC), lambda i, j: (0, 0)),
            pl.BlockSpec((batch, sb, cb, fh, fw),
                         lambda i, j: (0, i, j, 0, 0)),
        ],
        out_specs=pl.BlockSpec((batch, sb, cb, fh, fw),
                               lambda i, j: (0, i, j, 0, 0)),
        out_shape=jax.ShapeDtypeStruct(f.shape, jnp.float32),
        compiler_params=pltpu.CompilerParams(
            dimension_semantics=("arbitrary", "arbitrary"),
        ),
    )(emb_w, a1, a2, f)
    return out


# speedup vs baseline: 1.0030x; 1.0030x over previous
import numpy as np
import jax
import jax.numpy as jnp
from jax import lax
from jax.experimental import pallas as pl
from jax.experimental.pallas import tpu as pltpu

_H_DISC = 4
_W_DISC = 4
_DISC = _H_DISC * _W_DISC
_SEQ_BLOCK = 8
_CH_BLOCK = 96


def _interp_matrix(n_in: int, n_out: int) -> np.ndarray:
    a = np.zeros((n_out, n_in), np.float64)
    s = n_in / n_out
    for i in range(n_out):
        x = (i + 0.5) * s - 0.5
        lo = int(np.floor(x))
        t = x - lo
        for idx, w in ((lo, 1.0 - t), (lo + 1, t)):
            a[i, min(max(idx, 0), n_in - 1)] += w
    return a.astype(np.float32)


def _upsample_kernel(emb_ref, a1_ref, a2_ref, f_ref, o_ref):
    sb, cb = _SEQ_BLOCK, f_ref.shape[2]
    e4 = emb_ref[...].reshape(sb, cb, _H_DISC, _W_DISC)
    t1 = lax.dot_general(e4, a1_ref[...], (((2,), (1,)), ((), ())))
    cont = lax.dot_general(t1, a2_ref[...], (((2,), (1,)), ((), ())))
    o_ref[...] = f_ref[...] + cont[None]


def kernel(f, emb_w):
    batch, seq, ch, fh, fw = f.shape

    a1 = jnp.asarray(_interp_matrix(_H_DISC, fh))
    a2 = jnp.asarray(_interp_matrix(_W_DISC, fw))

    sb = _SEQ_BLOCK
    cb = min(_CH_BLOCK, ch)
    grid = (batch, seq // sb, ch // cb)

    out = pl.pallas_call(
        _upsample_kernel,
        grid=grid,
        in_specs=[
            pl.BlockSpec((sb, cb * _DISC), lambda b, j, c: (j, c)),
            pl.BlockSpec((fh, _H_DISC), lambda b, j, c: (0, 0)),
            pl.BlockSpec((fw, _W_DISC), lambda b, j, c: (0, 0)),
            pl.BlockSpec((1, sb, cb, fh, fw),
                         lambda b, j, c: (b, j, c, 0, 0)),
        ],
        out_specs=pl.BlockSpec((1, sb, cb, fh, fw),
                               lambda b, j, c: (b, j, c, 0, 0)),
        out_shape=jax.ShapeDtypeStruct(f.shape, jnp.float32),
        compiler_params=pltpu.CompilerParams(
            dimension_semantics=("arbitrary", "arbitrary", "arbitrary"),
        ),
    )(emb_w, a1, a2, f)
    return out


# speedup vs baseline: 13.4061x; 13.3665x over previous
import numpy as np
import jax
import jax.numpy as jnp
from jax.experimental import pallas as pl
from jax.experimental.pallas import tpu as pltpu

_H_DISC = 4
_W_DISC = 4
_DISC = _H_DISC * _W_DISC
_CH_BLOCK = 8


def _interp_matrix(n_in: int, n_out: int) -> np.ndarray:
    a = np.zeros((n_out, n_in), np.float64)
    s = n_in / n_out
    for i in range(n_out):
        x = (i + 0.5) * s - 0.5
        lo = int(np.floor(x))
        t = x - lo
        for idx, w in ((lo, 1.0 - t), (lo + 1, t)):
            a[i, min(max(idx, 0), n_in - 1)] += w
    return a.astype(np.float32)


def _upsample_kernel(emb_ref, k_ref, f_ref, o_ref):
    batch, cb, fh, fw, seq = f_ref.shape
    et = emb_ref[...].T
    k = k_ref[...]
    for c in range(cb):
        ec = et[c * _DISC:(c + 1) * _DISC, :]
        m = jnp.dot(k, ec, preferred_element_type=jnp.float32)
        m3 = m.reshape(fh, fw, seq)
        for b in range(batch):
            o_ref[b, c] = f_ref[b, c] + m3


def kernel(f, emb_w):
    batch, seq, ch, fh, fw = f.shape
    hw = fh * fw

    a1 = _interp_matrix(_H_DISC, fh)
    a2 = _interp_matrix(_W_DISC, fw)
    kmat = jnp.asarray(np.kron(a1, a2))

    ft = jnp.transpose(f, (0, 2, 3, 4, 1))

    cb = _CH_BLOCK
    grid = (ch // cb,)

    out_t = pl.pallas_call(
        _upsample_kernel,
        grid=grid,
        in_specs=[
            pl.BlockSpec((seq, cb * _DISC), lambda i: (0, i)),
            pl.BlockSpec((hw, _DISC), lambda i: (0, 0)),
            pl.BlockSpec((batch, cb, fh, fw, seq), lambda i: (0, i, 0, 0, 0)),
        ],
        out_specs=pl.BlockSpec((batch, cb, fh, fw, seq),
                               lambda i: (0, i, 0, 0, 0)),
        out_shape=jax.ShapeDtypeStruct((batch, ch, fh, fw, seq), jnp.float32),
        compiler_params=pltpu.CompilerParams(
            dimension_semantics=("arbitrary",),
        ),
    )(emb_w, kmat, ft)
    return jnp.transpose(out_t, (0, 4, 1, 2, 3))
